# baseline (speedup 1.0000x reference)
import functools

import jax
import jax.numpy as jnp
from jax import lax
from jax.experimental import pallas as pl
from jax.experimental.pallas import tpu as pltpu
from jax.experimental.pallas import tpu_sc as plsc

B, L, DIM = 4096, 200, 64
LN = 16
NCH = (L + LN - 1) // LN
LAST_OFF = L - LN


def _sc_pool(x, lengths, emb):
    info = plsc.get_sparse_core_info()
    nc, ns = info.num_cores, info.num_subcores
    nw = nc * ns
    bw = B // nw

    mesh = plsc.VectorSubcoreMesh(core_axis_name="c", subcore_axis_name="s")

    @functools.partial(
        pl.kernel,
        mesh=mesh,
        compiler_params=pltpu.CompilerParams(use_tc_tiling_on_sc=False),
        out_type=jax.ShapeDtypeStruct((B, DIM), jnp.float32),
        scratch_types=[
            pltpu.VMEM((bw, L), jnp.int32),
            pltpu.VMEM((bw + LN,), jnp.int32),
            pltpu.VMEM((2, NCH, LN, DIM), jnp.float32),
            pltpu.VMEM((bw, DIM), jnp.float32),
            pltpu.SemaphoreType.DMA,
        ],
    )
    def pool(x_hbm, len_hbm, emb_hbm, out_hbm, xb, lens, gbuf, ob, sem):
        wid = lax.axis_index("s") * nc + lax.axis_index("c")
        base = wid * bw
        pltpu.sync_copy(x_hbm.at[pl.ds(base, bw), :], xb)
        pltpu.sync_copy(len_hbm.at[pl.ds(base, bw)], lens.at[pl.ds(0, bw)])

        def length_of(i):
            return lens[pl.ds(i, LN)][0]

        def chunk_copy(i, slot, j):
            off = jnp.minimum(j * LN, LAST_OFF)
            return pltpu.make_async_copy(
                emb_hbm.at[xb.at[i, pl.ds(off, LN)]], gbuf.at[slot, j], sem
            )

        def fire(i, slot):
            ln = length_of(i)
            nch = (ln + (LN - 1)) // LN

            def go(j, c):
                chunk_copy(i, slot, j).start()
                return c

            lax.fori_loop(0, nch, go, 0)

        def drain(i, slot, ln):
            nch = (ln + (LN - 1)) // LN
            z = jnp.zeros((LN,), jnp.float32)

            def full_chunk(j, acc):
                chunk_copy(i, slot, j).wait()
                a0, a1, a2, a3 = acc
                for r in range(LN):
                    a0 = a0 + gbuf[slot, j, r, pl.ds(0, LN)]
                    a1 = a1 + gbuf[slot, j, r, pl.ds(LN, LN)]
                    a2 = a2 + gbuf[slot, j, r, pl.ds(2 * LN, LN)]
                    a3 = a3 + gbuf[slot, j, r, pl.ds(3 * LN, LN)]
                return (a0, a1, a2, a3)

            acc = lax.fori_loop(0, nch - 1, full_chunk, (z, z, z, z))

            last = nch - 1
            chunk_copy(i, slot, last).wait()
            off = jnp.minimum(last * LN, LAST_OFF)
            lo = last * LN
            acc = list(acc)
            for r in range(LN):
                pos = off + r
                gate = jnp.logical_and(pos >= lo, pos < ln).astype(jnp.float32)
                gv = lax.broadcast_in_dim(gate, (LN,), ())
                for g in range(4):
                    acc[g] = acc[g] + gbuf[slot, last, r, pl.ds(g * LN, LN)] * gv

            lnv = lax.broadcast_in_dim(ln.astype(jnp.float32), (LN,), ())
            for g in range(4):
                ob[i, pl.ds(g * LN, LN)] = acc[g] / lnv

        fire(0, 0)

        def example(i, c):
            ln = length_of(i)

            @pl.when(i < bw - 1)
            def _():
                fire(i + 1, (i + 1) % 2)

            drain(i, i % 2, ln)
            return c

        lax.fori_loop(0, bw, example, 0)
        pltpu.sync_copy(ob, out_hbm.at[pl.ds(base, bw), :])

    return pool(x, lengths, emb)


def _mlp(avg, W1, b1, W2, b2):
    def body(a_ref, w1_ref, b1_ref, w2_ref, b2_ref, o_ref):
        a = a_ref[:, :]
        h = lax.dot_general(a, w1_ref[:, :], (((1,), (1,)), ((), ())),
                            preferred_element_type=jnp.float32)
        h = jnp.maximum(h + b1_ref[:][None, :], 0.0)
        lg = lax.dot_general(h, w2_ref[:, :], (((1,), (1,)), ((), ())),
                             preferred_element_type=jnp.float32)
        lg = lg + b2_ref[:][None, :]
        m = jnp.max(lg, axis=1, keepdims=True)
        s = jnp.log(jnp.sum(jnp.exp(lg - m), axis=1, keepdims=True)) + m
        o_ref[:, :] = lg - s

    return pl.pallas_call(
        body,
        out_shape=jax.ShapeDtypeStruct((B, W2.shape[0]), jnp.float32),
    )(avg, W1, b1, W2, b2)


def kernel(x, lengths, emb, W1, b1, W2, b2):
    x = x.astype(jnp.int32)
    lengths = lengths.astype(jnp.int32)
    one = (1 - (lengths[0] >> 30)).astype(jnp.float32)
    avg = _sc_pool(x, lengths, emb * one)
    return _mlp(avg, W1, b1, W2, b2)


# speedup vs baseline: 1.1241x; 1.1241x over previous
import functools

import jax
import jax.numpy as jnp
from jax import lax
from jax.experimental import pallas as pl
from jax.experimental.pallas import tpu as pltpu
from jax.experimental.pallas import tpu_sc as plsc

B, L, DIM = 4096, 200, 64
LN = 16
NCH = (L + LN - 1) // LN
LAST_OFF = L - LN

_PERM = (
    list(range(0, 32, 2)) + list(range(1, 32, 2))
    + list(range(32, 64, 2)) + list(range(33, 64, 2))
)


def _sc_pool(x, lengths, emb):
    info = plsc.get_sparse_core_info()
    nc, ns = info.num_cores, info.num_subcores
    nw = nc * ns
    bw = B // nw

    mesh = plsc.VectorSubcoreMesh(core_axis_name="c", subcore_axis_name="s")

    @functools.partial(
        pl.kernel,
        mesh=mesh,
        compiler_params=pltpu.CompilerParams(
            use_tc_tiling_on_sc=False, needs_layout_passes=False
        ),
        out_type=jax.ShapeDtypeStruct((B, DIM), jnp.float32),
        scratch_types=[
            pltpu.VMEM((bw, L), jnp.int32),
            pltpu.VMEM((bw + LN,), jnp.int32),
            pltpu.VMEM((2, NCH, LN, DIM), jnp.bfloat16),
            pltpu.VMEM((bw, DIM), jnp.float32),
            pltpu.SemaphoreType.DMA,
        ],
    )
    def pool(x_hbm, len_hbm, emb_hbm, out_hbm, xb, lens, gbuf, ob, sem):
        wid = lax.axis_index("s") * nc + lax.axis_index("c")
        base = wid * bw
        pltpu.sync_copy(x_hbm.at[pl.ds(base, bw), :], xb)
        pltpu.sync_copy(len_hbm.at[pl.ds(base, bw)], lens.at[pl.ds(0, bw)])

        def length_of(i):
            return lens[pl.ds(i, LN)][0]

        def chunk_copy(i, slot, j):
            off = jnp.minimum(j * LN, LAST_OFF)
            return pltpu.make_async_copy(
                emb_hbm.at[xb.at[i, pl.ds(off, LN)]], gbuf.at[slot, j], sem
            )

        def fire(i, slot):
            ln = length_of(i)
            nch = (ln + (LN - 1)) // LN

            def go(j, c):
                chunk_copy(i, slot, j).start()
                return c

            lax.fori_loop(0, nch, go, 0)

        def row_halves(slot, j, r):
            b0 = gbuf[slot, j, r, pl.ds(0, 2 * LN)]
            b1 = gbuf[slot, j, r, pl.ds(2 * LN, 2 * LN)]
            e0, o0 = plsc.unpack(b0, format=plsc.PackFormat.INTERLEAVED)
            e1, o1 = plsc.unpack(b1, format=plsc.PackFormat.INTERLEAVED)
            return (e0, o0, e1, o1)

        def drain(i, slot, ln):
            nch = (ln + (LN - 1)) // LN
            z = jnp.zeros((LN,), jnp.float32)

            def full_chunk(j, acc):
                chunk_copy(i, slot, j).wait()
                a0, a1, a2, a3 = acc
                for r in range(LN):
                    h = row_halves(slot, j, r)
                    a0 = a0 + h[0]
                    a1 = a1 + h[1]
                    a2 = a2 + h[2]
                    a3 = a3 + h[3]
                return (a0, a1, a2, a3)

            acc = lax.fori_loop(0, nch - 1, full_chunk, (z, z, z, z))

            last = nch - 1
            chunk_copy(i, slot, last).wait()
            off = jnp.minimum(last * LN, LAST_OFF)
            lo = last * LN
            acc = list(acc)
            for r in range(LN):
                pos = off + r
                gate = jnp.logical_and(pos >= lo, pos < ln).astype(jnp.float32)
                gv = lax.broadcast_in_dim(gate, (LN,), ())
                h = row_halves(slot, last, r)
                for g in range(4):
                    acc[g] = acc[g] + h[g] * gv

            lnv = lax.broadcast_in_dim(ln.astype(jnp.float32), (LN,), ())
            for g in range(4):
                ob[i, pl.ds(g * LN, LN)] = acc[g] / lnv

        fire(0, 0)

        def example(i, c):
            ln = length_of(i)

            @pl.when(i < bw - 1)
            def _():
                fire(i + 1, (i + 1) % 2)

            drain(i, i % 2, ln)
            return c

        lax.fori_loop(0, bw, example, 0)
        pltpu.sync_copy(ob, out_hbm.at[pl.ds(base, bw), :])

    return pool(x, lengths, emb)


def _mlp(avg, W1, b1, W2, b2):
    def body(a_ref, w1_ref, b1_ref, w2_ref, b2_ref, o_ref):
        a = a_ref[:, :]
        h = lax.dot_general(a, w1_ref[:, :], (((1,), (1,)), ((), ())),
                            preferred_element_type=jnp.float32)
        h = jnp.maximum(h + b1_ref[:][None, :], 0.0)
        lg = lax.dot_general(h, w2_ref[:, :], (((1,), (1,)), ((), ())),
                             preferred_element_type=jnp.float32)
        lg = lg + b2_ref[:][None, :]
        m = jnp.max(lg, axis=1, keepdims=True)
        s = jnp.log(jnp.sum(jnp.exp(lg - m), axis=1, keepdims=True)) + m
        o_ref[:, :] = lg - s

    return pl.pallas_call(
        body,
        out_shape=jax.ShapeDtypeStruct((B, W2.shape[0]), jnp.float32),
    )(avg, W1, b1, W2, b2)


def kernel(x, lengths, emb, W1, b1, W2, b2):
    x = x.astype(jnp.int32)
    lengths = lengths.astype(jnp.int32)
    avg = _sc_pool(x, lengths, emb.astype(jnp.bfloat16))
    w1p = W1[:, jnp.array(_PERM)]
    return _mlp(avg, w1p, b1, W2, b2)


# speedup vs baseline: 1.5344x; 1.3650x over previous
import functools

import jax
import jax.numpy as jnp
from jax import lax
from jax.experimental import pallas as pl
from jax.experimental.pallas import tpu as pltpu
from jax.experimental.pallas import tpu_sc as plsc

B, L, DIM = 4096, 200, 64
LN = 16
NCH = (L + LN - 1) // LN
LAST_OFF = L - LN


def _sc_pool(x, lengths, emb):
    info = plsc.get_sparse_core_info()
    nc, ns = info.num_cores, info.num_subcores
    nw = nc * ns
    bw = B // nw

    mesh = plsc.VectorSubcoreMesh(core_axis_name="c", subcore_axis_name="s")

    @functools.partial(
        pl.kernel,
        mesh=mesh,
        compiler_params=pltpu.CompilerParams(use_tc_tiling_on_sc=False),
        out_type=jax.ShapeDtypeStruct((B, DIM), jnp.float32),
        scratch_types=[
            pltpu.VMEM((bw, L), jnp.int32),
            pltpu.VMEM((bw + LN,), jnp.int32),
            pltpu.VMEM((2, NCH, LN, 2 * DIM), jnp.float32),
            pltpu.VMEM((bw, DIM), jnp.float32),
            pltpu.SemaphoreType.DMA,
        ],
    )
    def pool(x_hbm, len_hbm, emb_hbm, out_hbm, xb, lens, gbuf, ob, sem):
        wid = lax.axis_index("s") * nc + lax.axis_index("c")
        base = wid * bw
        pltpu.sync_copy(x_hbm.at[pl.ds(base, bw), :], xb)
        pltpu.sync_copy(len_hbm.at[pl.ds(base, bw)], lens.at[pl.ds(0, bw)])

        def length_of(i):
            return lens[pl.ds(i, LN)][0]

        def chunk_copy(i, slot, j):
            off = jnp.minimum(j * LN, LAST_OFF)
            return pltpu.make_async_copy(
                emb_hbm.at[xb.at[i, pl.ds(off, LN)]], gbuf.at[slot, j], sem
            )

        def fire(i, slot):
            ln = length_of(i)
            nch = (ln + (LN - 1)) // LN

            def go(j, c):
                chunk_copy(i, slot, j).start()
                return c

            lax.fori_loop(0, nch, go, 0)

        def drain(i, slot, ln):
            nch = (ln + (LN - 1)) // LN
            z = jnp.zeros((LN,), jnp.float32)

            def full_chunk(j, acc):
                chunk_copy(i, slot, j).wait()
                a0, a1, a2, a3 = acc
                for r in range(LN):
                    a0 = a0 + gbuf[slot, j, r, pl.ds(0, LN)]
                    a1 = a1 + gbuf[slot, j, r, pl.ds(LN, LN)]
                    a2 = a2 + gbuf[slot, j, r, pl.ds(2 * LN, LN)]
                    a3 = a3 + gbuf[slot, j, r, pl.ds(3 * LN, LN)]
                return (a0, a1, a2, a3)

            acc = lax.fori_loop(0, nch - 1, full_chunk, (z, z, z, z))

            last = nch - 1
            chunk_copy(i, slot, last).wait()
            off = jnp.minimum(last * LN, LAST_OFF)
            lo = last * LN
            acc = list(acc)
            for r in range(LN):
                pos = off + r
                gate = jnp.logical_and(pos >= lo, pos < ln).astype(jnp.float32)
                gv = lax.broadcast_in_dim(gate, (LN,), ())
                for g in range(4):
                    acc[g] = acc[g] + gbuf[slot, last, r, pl.ds(g * LN, LN)] * gv

            lnv = lax.broadcast_in_dim(ln.astype(jnp.float32), (LN,), ())
            for g in range(4):
                ob[i, pl.ds(g * LN, LN)] = acc[g] / lnv

        fire(0, 0)

        def example(i, c):
            ln = length_of(i)

            @pl.when(i < bw - 1)
            def _():
                fire(i + 1, (i + 1) % 2)

            drain(i, i % 2, ln)
            return c

        lax.fori_loop(0, bw, example, 0)
        pltpu.sync_copy(ob, out_hbm.at[pl.ds(base, bw), :])

    return pool(x, lengths, emb)


def _mlp(avg, W1, b1, W2, b2):
    def body(a_ref, w1_ref, b1_ref, w2_ref, b2_ref, o_ref):
        a = a_ref[:, :]
        h = lax.dot_general(a, w1_ref[:, :], (((1,), (1,)), ((), ())),
                            preferred_element_type=jnp.float32)
        h = jnp.maximum(h + b1_ref[:][None, :], 0.0)
        lg = lax.dot_general(h, w2_ref[:, :], (((1,), (1,)), ((), ())),
                             preferred_element_type=jnp.float32)
        lg = lg + b2_ref[:][None, :]
        m = jnp.max(lg, axis=1, keepdims=True)
        s = jnp.log(jnp.sum(jnp.exp(lg - m), axis=1, keepdims=True)) + m
        o_ref[:, :] = lg - s

    return pl.pallas_call(
        body,
        out_shape=jax.ShapeDtypeStruct((B, W2.shape[0]), jnp.float32),
    )(avg, W1, b1, W2, b2)


def kernel(x, lengths, emb, W1, b1, W2, b2):
    x = x.astype(jnp.int32)
    lengths = lengths.astype(jnp.int32)
    avg = _sc_pool(x, lengths, jnp.pad(emb, ((0, 0), (0, DIM))))
    return _mlp(avg, W1, b1, W2, b2)
